# baseline (speedup 1.0000x reference)
import jax
import jax.numpy as jnp
from jax import lax
from jax.experimental import pallas as pl
from jax.experimental.pallas import tpu as pltpu
from jax.experimental.pallas import tpu_sc as plsc

N, P2, W2, CKV, TOPK = 8, 64, 64, 192, 4
ROW = W2 * CKV
OUT_ROWS = N * P2 * TOPK
NC, NS, LANES = 2, 16, 16
NW = NC * NS
RPW = OUT_ROWS // NW
CH = 4
NCH = RPW // CH


def _body(kv_hbm, idx_hbm, out_hbm, idx_v, bufs, sem0, sem1):
    wid = lax.axis_index("s") * NC + lax.axis_index("c")
    base = wid * RPW
    b = base // (P2 * TOPK)

    pltpu.sync_copy(idx_hbm.at[wid], idx_v)

    sems = (sem0, sem1)
    kv_b = kv_hbm.at[b]

    def gather_start(chunk):
        buf = chunk % 2
        src = kv_b.at[idx_v.at[chunk]]
        return pltpu.async_copy(src, bufs.at[buf], sems[buf])

    handles = [None, None]
    handles[0] = gather_start(0)
    for c in range(NCH):
        if c + 1 < NCH:
            handles[(c + 1) % 2] = gather_start(c + 1)
        handles[c % 2].wait()
        pltpu.sync_copy(bufs.at[c % 2], out_hbm.at[pl.ds(base + c * CH, CH)])


@jax.jit
def _gather(kv3, idx3):
    mesh = plsc.VectorSubcoreMesh(
        core_axis_name="c", subcore_axis_name="s", num_cores=NC, num_subcores=NS
    )
    return pl.kernel(
        _body,
        out_type=jax.ShapeDtypeStruct((OUT_ROWS, ROW), jnp.float32),
        mesh=mesh,
        scratch_types=[
            pltpu.VMEM((NCH, CH), jnp.int32),
            pltpu.VMEM((2, CH, ROW), jnp.float32),
            pltpu.SemaphoreType.DMA,
            pltpu.SemaphoreType.DMA,
        ],
    )(kv3, idx3)


def kernel(kv, r_idx):
    kv3 = kv.reshape(N, P2, ROW)
    idx3 = r_idx.reshape(NW, NCH, CH)
    out = _gather(kv3, idx3)
    return out.reshape(N, P2, TOPK, W2, CKV)


# speedup vs baseline: 2.0787x; 2.0787x over previous
import jax
import jax.numpy as jnp
from jax import lax
from jax.experimental import pallas as pl
from jax.experimental.pallas import tpu as pltpu
from jax.experimental.pallas import tpu_sc as plsc

N, P2, W2, CKV, TOPK = 8, 64, 64, 192, 4
NC, NS, LANES = 2, 16, 16
NW = NC * NS
IPW = N * P2 // NW
WPB = P2 // IPW
SPW = IPW * TOPK
NBUF = 4
LOOKAHEAD = 2


def _body(kv_hbm, idx_hbm, out_hbm, idx_v, bufs, gsems, ssems):
    wid = lax.axis_index("s") * NC + lax.axis_index("c")
    b = wid // WPB
    i0 = (wid % WPB) * IPW

    pltpu.sync_copy(idx_hbm.at[pl.ds(wid * SPW, SPW)], idx_v)

    kv_b = kv_hbm.at[b]
    out_b = out_hbm.at[b]

    groups = [idx_v[pl.ds(g * LANES, LANES)] for g in range(SPW // LANES)]

    def slab_index(s):
        return groups[s // LANES][s % LANES]

    def gather_start(s):
        return pltpu.async_copy(
            kv_b.at[slab_index(s)], bufs.at[s % NBUF], gsems[s % NBUF]
        )

    def scatter_start(s):
        dst = out_b.at[i0 + s // TOPK].at[s % TOPK]
        return pltpu.async_copy(bufs.at[s % NBUF], dst, ssems[s % NBUF])

    gh = [None] * NBUF
    sh = [None] * NBUF
    for s in range(LOOKAHEAD):
        gh[s % NBUF] = gather_start(s)
    for s in range(SPW):
        if s + LOOKAHEAD < SPW:
            gh[(s + LOOKAHEAD) % NBUF] = gather_start(s + LOOKAHEAD)
        gh[s % NBUF].wait()
        sh[s % NBUF] = scatter_start(s)
        if s - (NBUF - LOOKAHEAD - 1) >= 0:
            w = s - (NBUF - LOOKAHEAD - 1)
            if sh[w % NBUF] is not None:
                sh[w % NBUF].wait()
                sh[w % NBUF] = None
    for h in sh:
        if h is not None:
            h.wait()


@jax.jit
def kernel(kv, r_idx):
    mesh = plsc.VectorSubcoreMesh(
        core_axis_name="c", subcore_axis_name="s", num_cores=NC, num_subcores=NS
    )
    idx_flat = r_idx.reshape(N * P2 * TOPK)
    return pl.kernel(
        _body,
        out_type=jax.ShapeDtypeStruct((N, P2, TOPK, W2, CKV), jnp.float32),
        mesh=mesh,
        scratch_types=[
            pltpu.VMEM((SPW,), jnp.int32),
            pltpu.VMEM((NBUF, W2, CKV), jnp.float32),
            [pltpu.SemaphoreType.DMA] * NBUF,
            [pltpu.SemaphoreType.DMA] * NBUF,
        ],
    )(kv, idx_flat)


# speedup vs baseline: 2.2239x; 1.0699x over previous
import jax
import jax.numpy as jnp
from jax import lax
from jax.experimental import pallas as pl
from jax.experimental.pallas import tpu as pltpu
from jax.experimental.pallas import tpu_sc as plsc

N, P2, W2, CKV, TOPK = 8, 64, 64, 192, 4
NC, NS, LANES = 2, 16, 16
NW = NC * NS
IPW = N * P2 // NW
WPB = P2 // IPW
SPW = IPW * TOPK
NBUF = 4
LOOKAHEAD = 2


def _body(kv_hbm, idx_hbm, out_hbm, idx_v, bufs, gsems, ssems):
    sid = lax.axis_index("s")
    wid = sid * NC + lax.axis_index("c")
    b = wid // WPB
    i0 = (wid % WPB) * IPW

    pltpu.sync_copy(idx_hbm.at[pl.ds(wid * SPW, SPW)], idx_v)

    kv_b = kv_hbm.at[b]
    out_b = out_hbm.at[b]
    my_bufs = bufs.at[sid]

    groups = [idx_v[pl.ds(g * LANES, LANES)] for g in range(SPW // LANES)]

    def slab_index(s):
        return groups[s // LANES][s % LANES]

    def gather_start(s):
        return pltpu.async_copy(
            kv_b.at[slab_index(s)], my_bufs.at[s % NBUF], gsems[s % NBUF]
        )

    def scatter_start(s):
        dst = out_b.at[i0 + s // TOPK].at[s % TOPK]
        return pltpu.async_copy(my_bufs.at[s % NBUF], dst, ssems[s % NBUF])

    gh = [None] * NBUF
    sh = [None] * NBUF
    for s in range(LOOKAHEAD):
        gh[s % NBUF] = gather_start(s)
    for s in range(SPW):
        if s + LOOKAHEAD < SPW:
            gh[(s + LOOKAHEAD) % NBUF] = gather_start(s + LOOKAHEAD)
        gh[s % NBUF].wait()
        sh[s % NBUF] = scatter_start(s)
        if s - (NBUF - LOOKAHEAD - 1) >= 0:
            w = s - (NBUF - LOOKAHEAD - 1)
            if sh[w % NBUF] is not None:
                sh[w % NBUF].wait()
                sh[w % NBUF] = None
    for h in sh:
        if h is not None:
            h.wait()


@jax.jit
def kernel(kv, r_idx):
    mesh = plsc.VectorSubcoreMesh(
        core_axis_name="c", subcore_axis_name="s", num_cores=NC, num_subcores=NS
    )
    idx_flat = r_idx.reshape(N * P2 * TOPK)
    return pl.kernel(
        _body,
        out_type=jax.ShapeDtypeStruct((N, P2, TOPK, W2, CKV), jnp.float32),
        mesh=mesh,
        scratch_types=[
            pltpu.VMEM((SPW,), jnp.int32),
            pltpu.VMEM_SHARED((NS, NBUF, W2, CKV), jnp.float32),
            [pltpu.SemaphoreType.DMA] * NBUF,
            [pltpu.SemaphoreType.DMA] * NBUF,
        ],
    )(kv, idx_flat)
